# baseline (speedup 1.0000x reference)
import functools

import jax
import jax.numpy as jnp
from jax import lax
from jax.experimental import pallas as pl
from jax.experimental.pallas import tpu as pltpu
from jax.experimental.pallas import tpu_sc as plsc

N = 10000
K = 32
F = 64
L = 16
NC, NS = 2, 16
NW = NC * NS
RPW = 320
NP = NW * RPW
CQ = 4
CR = CQ * K
NCH = RPW // CQ
FV = F // L


def _dense_body(x_ref, w_ref, b_ref, o_ref):
    y = jnp.dot(x_ref[...], w_ref[...], preferred_element_type=jnp.float32)
    o_ref[...] = jnp.maximum(y + b_ref[...], 0.0)


def _dense_relu(x, w, b):
    bm = 1024
    din = x.shape[1]
    return pl.pallas_call(
        _dense_body,
        grid=(NP // bm,),
        in_specs=[
            pl.BlockSpec((bm, din), lambda i: (i, 0)),
            pl.BlockSpec((din, F), lambda i: (0, 0)),
            pl.BlockSpec((1, F), lambda i: (0, 0)),
        ],
        out_specs=pl.BlockSpec((bm, F), lambda i: (i, 0)),
        out_shape=jax.ShapeDtypeStruct((NP, F), jnp.float32),
    )(x, w, b.reshape(1, F))


_MESH = plsc.VectorSubcoreMesh(core_axis_name="c", subcore_axis_name="s")


@functools.partial(
    pl.kernel,
    out_type=jax.ShapeDtypeStruct((NP, 2 * F), jnp.float32),
    mesh=_MESH,
    scratch_types=[
        pltpu.VMEM((NCH, CR), jnp.int32),
        pltpu.VMEM((RPW, F), jnp.float32),
        pltpu.VMEM((4, CR, F), jnp.float32),
        pltpu.VMEM((RPW, 2 * F), jnp.float32),
        pltpu.SemaphoreType.DMA,
        pltpu.SemaphoreType.DMA,
        pltpu.SemaphoreType.DMA,
        pltpu.SemaphoreType.DMA,
    ],
    compiler_params=pltpu.CompilerParams(use_tc_tiling_on_sc=False),
)
def _sc_accum(table_hbm, idx_hbm, out_hbm, idx_v, feat_v, gbuf, out_v,
              sem0, sem1, sem2, sem3):
    NB = 4
    wid = lax.axis_index("s") * NC + lax.axis_index("c")
    base = wid * RPW
    pltpu.sync_copy(idx_hbm.at[wid], idx_v)
    pltpu.sync_copy(table_hbm.at[pl.ds(base, RPW)], feat_v)
    sems = (sem0, sem1, sem2, sem3)
    for b in range(NB):
        pltpu.async_copy(table_hbm.at[idx_v.at[b]], gbuf.at[b], sems[b])

    @pl.loop(0, NCH, step=NB)
    def _ring(g):
        for b in range(NB):
            ch = g + b
            sem = sems[b]
            pltpu.make_async_copy(
                table_hbm.at[idx_v.at[ch]], gbuf.at[b], sem).wait()
            for qi in range(CQ):
                row0 = qi * K
                sums = [gbuf[b, row0, pl.ds(f * L, L)] for f in range(FV)]
                maxs = list(sums)
                for n in range(1, K):
                    for f in range(FV):
                        v = gbuf[b, row0 + n, pl.ds(f * L, L)]
                        sums[f] = sums[f] + v
                        maxs[f] = jnp.maximum(maxs[f], v)
                qrow = ch * CQ + qi
                for f in range(FV):
                    fv = feat_v[qrow, pl.ds(f * L, L)]
                    out_v[qrow, pl.ds(f * L, L)] = sums[f] * (1.0 / K) - fv
                    out_v[qrow, pl.ds(F + f * L, L)] = maxs[f] - fv

            @pl.when(ch + NB < NCH)
            def _prefetch(_b=b, _ch=ch, _sem=sem):
                pltpu.async_copy(
                    table_hbm.at[idx_v.at[_ch + NB]], gbuf.at[_b], _sem)

    pltpu.sync_copy(out_v, out_hbm.at[pl.ds(base, RPW)])


def kernel(x, neighbour_indices, W0, b0, W1, b1, W2, b2):
    xp = jnp.pad(x, ((0, NP - N), (0, 0)))
    idxp = jnp.pad(neighbour_indices.astype(jnp.int32), ((0, NP - N), (0, 0)))
    idx3 = idxp.reshape(NW, NCH, CR)
    prev = xp
    outs = []
    for (W, b) in ((W0, b0), (W1, b1), (W2, b2)):
        feat = _dense_relu(prev, W, b)
        o = _sc_accum(feat, idx3)
        outs.append(o[:N])
        prev = o
    return jnp.concatenate(outs + [x], axis=1)


# speedup vs baseline: 1.7965x; 1.7965x over previous
import functools

import jax
import jax.numpy as jnp
import numpy as np
from jax import lax
from jax.experimental import pallas as pl
from jax.experimental.pallas import tpu as pltpu
from jax.experimental.pallas import tpu_sc as plsc

N = 10000
K = 32
F = 64
L = 16
NC, NS = 2, 16
NW = NC * NS
RPW = 320
NP = NW * RPW
CQ = 4
CR = CQ * K
NCH = RPW // CQ
FV = F // L


_SIG = np.concatenate(
    [g * 32 + np.stack([np.arange(16), np.arange(16, 32)], 1).reshape(-1)
     for g in (0, 1)]).astype(np.int32)


def _dense_body(x_ref, w_ref, ws_ref, b_ref, bs_ref, of_ref, ob_ref):
    xv = x_ref[...]
    yf = jnp.dot(xv, w_ref[...], preferred_element_type=jnp.float32)
    of_ref[...] = jnp.maximum(yf + b_ref[...], 0.0)
    yb = jnp.dot(xv, ws_ref[...], preferred_element_type=jnp.float32)
    ob_ref[...] = jnp.maximum(yb + bs_ref[...], 0.0).astype(jnp.bfloat16)


def _dense_relu(x, w, b):
    bm = 1024
    din = x.shape[1]
    return pl.pallas_call(
        _dense_body,
        grid=(NP // bm,),
        in_specs=[
            pl.BlockSpec((bm, din), lambda i: (i, 0)),
            pl.BlockSpec((din, F), lambda i: (0, 0)),
            pl.BlockSpec((din, F), lambda i: (0, 0)),
            pl.BlockSpec((1, F), lambda i: (0, 0)),
            pl.BlockSpec((1, F), lambda i: (0, 0)),
        ],
        out_specs=[
            pl.BlockSpec((bm, F), lambda i: (i, 0)),
            pl.BlockSpec((bm, F), lambda i: (i, 0)),
        ],
        out_shape=[
            jax.ShapeDtypeStruct((NP, F), jnp.float32),
            jax.ShapeDtypeStruct((NP, F), jnp.bfloat16),
        ],
    )(x, w, w[:, _SIG], b.reshape(1, F), b[_SIG].reshape(1, F))


_MESH = plsc.VectorSubcoreMesh(core_axis_name="c", subcore_axis_name="s")


@functools.partial(
    pl.kernel,
    out_type=jax.ShapeDtypeStruct((NP, 2 * F), jnp.float32),
    mesh=_MESH,
    scratch_types=[
        pltpu.VMEM((NCH, CR), jnp.int32),
        pltpu.VMEM((RPW, F), jnp.float32),
        pltpu.VMEM((4, CR, F), jnp.bfloat16),
        pltpu.VMEM((RPW, 2 * F), jnp.float32),
        pltpu.SemaphoreType.DMA,
        pltpu.SemaphoreType.DMA,
        pltpu.SemaphoreType.DMA,
        pltpu.SemaphoreType.DMA,
    ],
    compiler_params=pltpu.CompilerParams(
        use_tc_tiling_on_sc=False, needs_layout_passes=False),
)
def _sc_accum(tablef_hbm, tableb_hbm, idx_hbm, out_hbm, idx_v, feat_v, gbuf,
              out_v, sem0, sem1, sem2, sem3):
    NB = 4
    wid = lax.axis_index("s") * NC + lax.axis_index("c")
    base = wid * RPW
    pltpu.sync_copy(idx_hbm.at[wid], idx_v)
    pltpu.sync_copy(tablef_hbm.at[pl.ds(base, RPW)], feat_v)
    sems = (sem0, sem1, sem2, sem3)
    for b in range(NB):
        pltpu.async_copy(tableb_hbm.at[idx_v.at[b]], gbuf.at[b], sems[b])

    @pl.loop(0, NCH, step=NB)
    def _ring(g):
        for b in range(NB):
            ch = g + b
            sem = sems[b]
            pltpu.make_async_copy(
                tableb_hbm.at[idx_v.at[ch]], gbuf.at[b], sem).wait()
            for qi in range(CQ):
                row0 = qi * K

                def _row(r, _b=b):
                    out = []
                    for g in range(2):
                        a0, a1 = plsc.unpack(
                            gbuf[_b, r, pl.ds(g * 32, 32)],
                            format=plsc.PackFormat.INTERLEAVED,
                            preferred_element_type=jnp.float32)
                        out += [a0, a1]
                    return out

                sums = _row(row0)
                maxs = list(sums)
                for n in range(1, K):
                    vals = _row(row0 + n)
                    for f in range(FV):
                        sums[f] = sums[f] + vals[f]
                        maxs[f] = jnp.maximum(maxs[f], vals[f])
                qrow = ch * CQ + qi
                for f in range(FV):
                    fv = feat_v[qrow, pl.ds(f * L, L)]
                    out_v[qrow, pl.ds(f * L, L)] = sums[f] * (1.0 / K) - fv
                    out_v[qrow, pl.ds(F + f * L, L)] = maxs[f] - fv

            @pl.when(ch + NB < NCH)
            def _prefetch(_b=b, _ch=ch, _sem=sem):
                pltpu.async_copy(
                    tableb_hbm.at[idx_v.at[_ch + NB]], gbuf.at[_b], _sem)

    pltpu.sync_copy(out_v, out_hbm.at[pl.ds(base, RPW)])


def kernel(x, neighbour_indices, W0, b0, W1, b1, W2, b2):
    xp = jnp.pad(x, ((0, NP - N), (0, 0)))
    idxp = jnp.pad(neighbour_indices.astype(jnp.int32), ((0, NP - N), (0, 0)))
    idx3 = idxp.reshape(NW, NCH, CR)
    prev = xp
    outs = []
    for (W, b) in ((W0, b0), (W1, b1), (W2, b2)):
        featf, featb = _dense_relu(prev, W, b)
        o = _sc_accum(featf, featb, idx3)
        outs.append(o[:N])
        prev = o
    return jnp.concatenate(outs + [x], axis=1)


# speedup vs baseline: 2.9511x; 1.6427x over previous
import functools

import jax
import jax.numpy as jnp
import numpy as np
from jax import lax
from jax.experimental import pallas as pl
from jax.experimental.pallas import tpu as pltpu
from jax.experimental.pallas import tpu_sc as plsc

N = 10000
K = 32
F = 64
L = 16
NC, NS = 2, 16
NP = 10240
QH = NP // NC
QB = 512
NCHQ = QH // QB
FT = F // NS


def _dense_body(wt_ref, x_ref, b_ref, o_ref):
    y = jnp.dot(wt_ref[...], x_ref[...], preferred_element_type=jnp.float32)
    o_ref[...] = jnp.maximum(y + b_ref[...], 0.0)


def _dense_relu_t(xt, w, b):
    bn = 2048
    din = xt.shape[0]
    return pl.pallas_call(
        _dense_body,
        grid=(NP // bn,),
        in_specs=[
            pl.BlockSpec((F, din), lambda i: (0, 0)),
            pl.BlockSpec((din, bn), lambda i: (0, i)),
            pl.BlockSpec((F, 1), lambda i: (0, 0)),
        ],
        out_specs=pl.BlockSpec((F, bn), lambda i: (0, i)),
        out_shape=jax.ShapeDtypeStruct((F, NP), jnp.float32),
    )(w.T, xt, b.reshape(F, 1))


_MESH = plsc.VectorSubcoreMesh(core_axis_name="c", subcore_axis_name="s")


@functools.partial(
    pl.kernel,
    out_type=jax.ShapeDtypeStruct((2 * F, NP), jnp.float32),
    mesh=_MESH,
    scratch_types=[
        pltpu.VMEM((FT, NP), jnp.float32),
        pltpu.VMEM((2, K, QB), jnp.int32),
        pltpu.VMEM((2, 2 * FT, QB), jnp.float32),
        pltpu.SemaphoreType.DMA,
        pltpu.SemaphoreType.DMA,
        pltpu.SemaphoreType.DMA,
        pltpu.SemaphoreType.DMA,
    ],
    compiler_params=pltpu.CompilerParams(
        use_tc_tiling_on_sc=False, needs_layout_passes=False),
)
def _sc_accum_t(tablet_hbm, idxt_hbm, outt_hbm, table_v, idx_v, out_v,
                isem0, isem1, osem0, osem1):
    sid = lax.axis_index("s")
    cid = lax.axis_index("c")
    qoff = cid * QH
    rbase = sid * FT
    pltpu.sync_copy(tablet_hbm.at[pl.ds(rbase, FT)], table_v)
    isems = (isem0, isem1)
    osems = (osem0, osem1)

    def _idx_src(ch):
        return idxt_hbm.at[:, pl.ds(qoff + ch * QB, QB)]

    def _out_dst(ch, half):
        return outt_hbm.at[pl.ds(half * F + rbase, FT),
                           pl.ds(qoff + ch * QB, QB)]

    for b in range(2):
        pltpu.async_copy(_idx_src(b), idx_v.at[b], isems[b])

    @pl.loop(0, NCHQ, step=2)
    def _chunk(g):
        for b in range(2):
            ch = g + b
            pltpu.make_async_copy(_idx_src(ch), idx_v.at[b], isems[b]).wait()

            @pl.when(ch >= 2)
            def _drain(_b=b, _ch=ch):
                for half in range(2):
                    pltpu.make_async_copy(
                        out_v.at[_b, pl.ds(half * FT, FT)],
                        _out_dst(_ch - 2, half), osems[_b]).wait()

            @pl.loop(0, QB // L)
            def _lane_group(lg, _b=b, _ch=ch):
                q0 = lg * L
                rows = [jnp.full((L,), f, jnp.int32) for f in range(FT)]
                iv0 = idx_v[_b, 0, pl.ds(q0, L)]
                sums = [plsc.load_gather(table_v, [rows[f], iv0])
                        for f in range(FT)]
                maxs = list(sums)
                for n in range(1, K):
                    iv = idx_v[_b, n, pl.ds(q0, L)]
                    for f in range(FT):
                        v = plsc.load_gather(table_v, [rows[f], iv])
                        sums[f] = sums[f] + v
                        maxs[f] = jnp.maximum(maxs[f], v)
                qg = qoff + _ch * QB + q0
                for f in range(FT):
                    fv = table_v[f, pl.ds(qg, L)]
                    out_v[_b, f, pl.ds(q0, L)] = sums[f] * (1.0 / K) - fv
                    out_v[_b, FT + f, pl.ds(q0, L)] = maxs[f] - fv

            for half in range(2):
                pltpu.async_copy(out_v.at[b, pl.ds(half * FT, FT)],
                                 _out_dst(ch, half), osems[b])

            @pl.when(ch + 2 < NCHQ)
            def _prefetch(_b=b, _ch=ch):
                pltpu.async_copy(_idx_src(_ch + 2), idx_v.at[_b], isems[_b])

    for b in range(2):
        for half in range(2):
            pltpu.make_async_copy(
                out_v.at[b, pl.ds(half * FT, FT)],
                _out_dst(NCHQ - 2 + b, half), osems[b]).wait()


def kernel(x, neighbour_indices, W0, b0, W1, b1, W2, b2):
    idxp = jnp.pad(neighbour_indices.astype(jnp.int32), ((0, NP - N), (0, 0)))
    idxt = idxp.T.reshape(K, NP)
    xt = jnp.pad(x.T, ((0, 0), (0, NP - N)))
    prevt = xt
    outs = []
    for (W, b) in ((W0, b0), (W1, b1), (W2, b2)):
        featt = _dense_relu_t(prevt, W, b)
        ot = _sc_accum_t(featt, idxt)
        outs.append(ot[:, :N])
        prevt = ot
    return jnp.concatenate(outs + [x.T[:, :N]], axis=0).T
